# baseline (speedup 1.0000x reference)
import functools

import jax
import jax.numpy as jnp
from jax import lax
from jax.experimental import pallas as pl
from jax.experimental.pallas import tpu as pltpu
from jax.experimental.pallas import tpu_sc as plsc

_L = 16
_CP = 4
_NB = 4
_NW = 32


def _sc_body(x_hbm, g_hbm, w_hbm, m_hbm, y_hbm, out_hbm,
             xb00, xb01, xb02, xb03, xb10, xb11, xb12, xb13,
             gbuf0, gbuf1, wtab, mtab, ytab,
             sem_tab, sem_in0, sem_in1, sem_out0, sem_out1):
    xbufs = ((xb00, xb01, xb02, xb03), (xb10, xb11, xb12, xb13))
    gbufs = (gbuf0, gbuf1)
    wid = lax.axis_index("s") * 2 + lax.axis_index("c")
    n_rows = x_hbm.shape[0]
    s_len = n_rows // _NB
    pos_per_w = s_len // _NW
    p_base = wid * pos_per_w
    n_chunks = pos_per_w // _CP

    wn = w_hbm.shape[0]
    mn = m_hbm.shape[0]
    yn = y_hbm.shape[0]
    d_g = g_hbm.shape[1]

    sems_in = (sem_in0, sem_in1)
    sems_out = (sem_out0, sem_out1)

    cw = pltpu.async_copy(w_hbm, wtab, sem_tab)
    cm = pltpu.async_copy(m_hbm, mtab, sem_tab)
    cy = pltpu.async_copy(y_hbm, ytab, sem_tab)

    def issue_in(g, sl):
        p0 = p_base + g * _CP
        sem = sems_in[sl]
        for b in range(_NB):
            pltpu.async_copy(
                x_hbm.at[pl.ds(b * s_len + p0, _CP)], xbufs[sl][b], sem
            )
        pltpu.async_copy(g_hbm.at[pl.ds(p0, _CP)], gbufs[sl], sem)

    def wait_in(sl):
        sem = sems_in[sl]
        for b in range(_NB):
            pltpu.make_async_copy(
                x_hbm.at[pl.ds(0, _CP)], xbufs[sl][b], sem
            ).wait()
        pltpu.make_async_copy(g_hbm.at[pl.ds(0, _CP)], gbufs[sl], sem).wait()

    def issue_out(g, sl):
        p0 = p_base + g * _CP
        sem = sems_out[sl]
        for b in range(_NB):
            pltpu.async_copy(
                xbufs[sl][b], out_hbm.at[pl.ds(b * s_len + p0, _CP)], sem
            )

    def wait_out(sl):
        for b in range(_NB):
            pltpu.make_async_copy(
                xbufs[sl][b], out_hbm.at[pl.ds(0, _CP)], sems_out[sl]
            ).wait()

    def compute(sl, p0):
        bufs = xbufs[sl]
        gb = gbufs[sl]

        @plsc.parallel_loop(0, _CP)
        def pos(j):
            s = p0 + j
            widx = s % wn
            midx = s % mn
            yidx = s % yn
            for piece, (tab, ridx) in enumerate(
                ((gb, j), (wtab, widx), (mtab, midx), (ytab, yidx))
            ):
                for k in range(d_g // _L):
                    pe_v = tab[ridx, pl.ds(k * _L, _L)]
                    col = pl.ds(piece * d_g + k * _L, _L)
                    for b in range(_NB):
                        bufs[b][j, col] = bufs[b][j, col] + pe_v

    issue_in(0, 0)
    cw.wait()
    cm.wait()
    cy.wait()

    def pair(p, _):
        for sl in (0, 1):
            g = 2 * p + sl
            nxt = g + 1
            nsl = 1 - sl

            @pl.when(jnp.logical_and(nxt < n_chunks, nxt >= 2))
            def _():
                wait_out(nsl)

            @pl.when(nxt < n_chunks)
            def _():
                issue_in(nxt, nsl)

            wait_in(sl)
            compute(sl, p_base + g * _CP)
            issue_out(g, sl)
        return 0

    lax.fori_loop(0, n_chunks // 2, pair, 0, unroll=False)
    wait_out(0)
    wait_out(1)


@jax.jit
def kernel(x, global_pe, week_pe, month_pe, year_pe):
    B, S, D = x.shape
    d_g = global_pe.shape[1]
    x2 = x.reshape(B * S, D)
    mesh = plsc.VectorSubcoreMesh(core_axis_name="c", subcore_axis_name="s")
    xbuf_t = pltpu.VMEM((_CP, D), jnp.float32)
    gbuf_t = pltpu.VMEM((_CP, d_g), jnp.float32)
    k = functools.partial(
        pl.kernel,
        mesh=mesh,
        out_type=jax.ShapeDtypeStruct((B * S, D), jnp.float32),
        scratch_types=[
            xbuf_t, xbuf_t, xbuf_t, xbuf_t,
            xbuf_t, xbuf_t, xbuf_t, xbuf_t,
            gbuf_t, gbuf_t,
            pltpu.VMEM(week_pe.shape, jnp.float32),
            pltpu.VMEM(month_pe.shape, jnp.float32),
            pltpu.VMEM(year_pe.shape, jnp.float32),
            pltpu.SemaphoreType.DMA,
            pltpu.SemaphoreType.DMA,
            pltpu.SemaphoreType.DMA,
            pltpu.SemaphoreType.DMA,
            pltpu.SemaphoreType.DMA,
        ],
    )(_sc_body)
    out = k(x2, global_pe, week_pe, month_pe, year_pe)
    return out.reshape(B, S, D)


# speedup vs baseline: 1.1950x; 1.1950x over previous
import functools

import jax
import jax.numpy as jnp
from jax import lax
from jax.experimental import pallas as pl
from jax.experimental.pallas import tpu as pltpu
from jax.experimental.pallas import tpu_sc as plsc

_L = 16
_CP = 8
_NB = 4
_NW = 32


def _sc_body(x_hbm, g_hbm, w_hbm, m_hbm, y_hbm, out_hbm,
             xb00, xb01, xb02, xb03, xb10, xb11, xb12, xb13,
             gslab, wtab, mtab, ybuf0, ybuf1, idxy,
             sem_tab, sem_in0, sem_in1, sem_out0, sem_out1):
    xbufs = ((xb00, xb01, xb02, xb03), (xb10, xb11, xb12, xb13))
    ybufs = (ybuf0, ybuf1)
    wid = lax.axis_index("s") * 2 + lax.axis_index("c")
    n_rows = x_hbm.shape[0]
    s_len = n_rows // _NB
    pos_per_w = s_len // _NW
    p_base = wid * pos_per_w
    n_chunks = pos_per_w // _CP

    wn = w_hbm.shape[0]
    mn = m_hbm.shape[0]
    yn = y_hbm.shape[0]
    d_g = g_hbm.shape[1]

    sems_in = (sem_in0, sem_in1)
    sems_out = (sem_out0, sem_out1)

    cw = pltpu.async_copy(w_hbm, wtab, sem_tab)
    cm = pltpu.async_copy(m_hbm, mtab, sem_tab)
    cg = pltpu.async_copy(g_hbm.at[pl.ds(p_base, pos_per_w)], gslab, sem_tab)

    def issue_in(g, sl):
        p0 = p_base + g * _CP
        sem = sems_in[sl]
        for b in range(_NB):
            pltpu.async_copy(
                x_hbm.at[pl.ds(b * s_len + p0, _CP)], xbufs[sl][b], sem
            )
        idxy[sl, pl.ds(0, _L)] = (lax.iota(jnp.int32, _L) + p0) % yn
        pltpu.async_copy(y_hbm.at[idxy.at[sl]], ybufs[sl], sem)

    def wait_in(sl):
        sem = sems_in[sl]
        for b in range(_NB):
            pltpu.make_async_copy(
                x_hbm.at[pl.ds(0, _CP)], xbufs[sl][b], sem
            ).wait()
        pltpu.make_async_copy(y_hbm.at[idxy.at[sl]], ybufs[sl], sem).wait()

    def issue_out(g, sl):
        p0 = p_base + g * _CP
        sem = sems_out[sl]
        for b in range(_NB):
            pltpu.async_copy(
                xbufs[sl][b], out_hbm.at[pl.ds(b * s_len + p0, _CP)], sem
            )

    def wait_out(sl):
        for b in range(_NB):
            pltpu.make_async_copy(
                xbufs[sl][b], out_hbm.at[pl.ds(0, _CP)], sems_out[sl]
            ).wait()

    def compute(sl, g):
        bufs = xbufs[sl]
        yb = ybufs[sl]
        p0 = p_base + g * _CP
        goff = g * _CP

        @plsc.parallel_loop(0, _CP)
        def pos(j):
            s = p0 + j
            widx = s % wn
            midx = s % mn
            for piece, (tab, ridx) in enumerate(
                ((gslab, goff + j), (wtab, widx), (mtab, midx), (yb, j))
            ):
                for k in range(d_g // _L):
                    pe_v = tab[ridx, pl.ds(k * _L, _L)]
                    col = pl.ds(piece * d_g + k * _L, _L)
                    for b in range(_NB):
                        bufs[b][j, col] = bufs[b][j, col] + pe_v

    issue_in(0, 0)
    cw.wait()
    cm.wait()
    cg.wait()

    def pair(p, _):
        for sl in (0, 1):
            g = 2 * p + sl
            nxt = g + 1
            nsl = 1 - sl

            @pl.when(jnp.logical_and(nxt < n_chunks, nxt >= 2))
            def _():
                wait_out(nsl)

            @pl.when(nxt < n_chunks)
            def _():
                issue_in(nxt, nsl)

            wait_in(sl)
            compute(sl, g)
            issue_out(g, sl)
        return 0

    lax.fori_loop(0, n_chunks // 2, pair, 0, unroll=False)
    wait_out(0)
    wait_out(1)


@jax.jit
def kernel(x, global_pe, week_pe, month_pe, year_pe):
    B, S, D = x.shape
    d_g = global_pe.shape[1]
    x2 = x.reshape(B * S, D)
    mesh = plsc.VectorSubcoreMesh(core_axis_name="c", subcore_axis_name="s")
    xbuf_t = pltpu.VMEM((_CP, D), jnp.float32)
    k = functools.partial(
        pl.kernel,
        mesh=mesh,
        out_type=jax.ShapeDtypeStruct((B * S, D), jnp.float32),
        scratch_types=[
            xbuf_t, xbuf_t, xbuf_t, xbuf_t,
            xbuf_t, xbuf_t, xbuf_t, xbuf_t,
            pltpu.VMEM((S // _NW, d_g), jnp.float32),
            pltpu.VMEM(week_pe.shape, jnp.float32),
            pltpu.VMEM(month_pe.shape, jnp.float32),
            pltpu.VMEM((_L, d_g), jnp.float32),
            pltpu.VMEM((_L, d_g), jnp.float32),
            pltpu.VMEM((2, _L), jnp.int32),
            pltpu.SemaphoreType.DMA,
            pltpu.SemaphoreType.DMA,
            pltpu.SemaphoreType.DMA,
            pltpu.SemaphoreType.DMA,
            pltpu.SemaphoreType.DMA,
        ],
    )(_sc_body)
    out = k(x2, global_pe, week_pe, month_pe, year_pe)
    return out.reshape(B, S, D)
